# baseline (speedup 1.0000x reference)
import functools

import jax
import jax.numpy as jnp
from jax import lax
from jax.experimental import pallas as pl
from jax.experimental.pallas import tpu as pltpu
from jax.experimental.pallas import tpu_sc as plsc

B = 1024
S = 50
OUT_W = 1100
RT = 138
CT = B // 128
L = 16
NC, NS = 2, 16
NW = NC * NS
CPW = B // NW
GROUPS = CPW // L

_mesh = plsc.VectorSubcoreMesh(core_axis_name="c", subcore_axis_name="s")


@functools.partial(
    pl.kernel,
    out_type=jax.ShapeDtypeStruct((RT, CT, 8, 128), jnp.float32),
    mesh=_mesh,
    scratch_types=[
        pltpu.VMEM((S, CPW), jnp.int32),
        pltpu.VMEM((RT, 8, CPW), jnp.float32),
        pltpu.SemaphoreType.DMA,
        pltpu.SemaphoreType.DMA,
        pltpu.SemaphoreType.DMA,
    ],
    compiler_params=pltpu.CompilerParams(
        use_tc_tiling_on_sc=False, needs_layout_passes=False
    ),
)
def _bow_kernel(idx_hbm, zeros_hbm, out_hbm, idx_v, hist_v, in_sem, z_sem,
                out_sem):
    wid = lax.axis_index("s") * NC + lax.axis_index("c")
    base = wid * CPW
    ct = base // 128
    ci0 = base % 128

    z_dma = pltpu.async_copy(zeros_hbm, hist_v, z_sem)
    in_dma = pltpu.async_copy(idx_hbm.at[:, pl.ds(base, CPW)], idx_v, in_sem)

    in_dma.wait()
    z_dma.wait()

    ones = jnp.ones((L,), jnp.float32)
    lanes = lax.iota(jnp.int32, L)
    for g in range(GROUPS):
        cols = lanes + g * L

        @plsc.parallel_loop(0, S, step=1, unroll=5)
        def _step(t):
            tok = idx_v[t, pl.ds(g * L, L)]
            r = jnp.maximum(tok - 1, 0)
            plsc.addupdate_scatter(
                hist_v, [r >> 3, r & 7, cols], ones, mask=tok > 0
            )

    HALF = RT // 2
    top = pltpu.async_copy(
        hist_v.at[pl.ds(0, HALF)],
        out_hbm.at[pl.ds(0, HALF), ct, :, pl.ds(ci0, CPW)],
        out_sem,
    )
    pltpu.sync_copy(
        hist_v.at[pl.ds(HALF, RT - HALF)],
        out_hbm.at[pl.ds(HALF, RT - HALF), ct, :, pl.ds(ci0, CPW)],
    )
    top.wait()


def kernel(inputs):
    zblock = jnp.zeros((RT, 8, CPW), jnp.float32)
    out4 = _bow_kernel(inputs.T, zblock)
    x = out4.transpose(0, 2, 1, 3).reshape(RT * 8, B)
    return x.T[:, :OUT_W]


# speedup vs baseline: 1.2337x; 1.2337x over previous
import functools

import jax
import jax.numpy as jnp
from jax import lax
from jax.experimental import pallas as pl
from jax.experimental.pallas import tpu as pltpu
from jax.experimental.pallas import tpu_sc as plsc

B = 1024
S = 50
OUT_W = 1100
RT = 138
CT = B // 128
L = 16
NC, NS = 2, 16
NW = NC * NS
CPW = B // NW
GROUPS = CPW // L

_mesh = plsc.VectorSubcoreMesh(core_axis_name="c", subcore_axis_name="s")


@functools.partial(
    pl.kernel,
    out_type=jax.ShapeDtypeStruct((RT, CT, 8, 128), jnp.float32),
    mesh=_mesh,
    scratch_types=[
        pltpu.VMEM((S, CPW), jnp.int32),
        pltpu.VMEM((RT, 8, CPW), jnp.float32),
        pltpu.SemaphoreType.DMA,
        pltpu.SemaphoreType.DMA,
    ],
    compiler_params=pltpu.CompilerParams(
        use_tc_tiling_on_sc=False, needs_layout_passes=False
    ),
)
def _bow_kernel(idx_hbm, out_hbm, idx_v, hist_v, in_sem, out_sem):
    wid = lax.axis_index("s") * NC + lax.axis_index("c")
    base = wid * CPW
    ct = base // 128
    ci0 = base % 128

    in_dma = pltpu.async_copy(idx_hbm.at[:, pl.ds(base, CPW)], idx_v, in_sem)

    zeros = jnp.zeros((L,), jnp.float32)

    @plsc.parallel_loop(0, RT * 8, step=1, unroll=8)
    def _zero(r):
        rt = r >> 3
        ri = r & 7
        hist_v[rt, ri, pl.ds(0, L)] = zeros
        hist_v[rt, ri, pl.ds(L, L)] = zeros

    in_dma.wait()

    ones = jnp.ones((L,), jnp.float32)
    lanes = lax.iota(jnp.int32, L)
    for g in range(GROUPS):
        cols = lanes + g * L

        @plsc.parallel_loop(0, S, step=1, unroll=5)
        def _step(t):
            tok = idx_v[t, pl.ds(g * L, L)]
            r = jnp.maximum(tok - 1, 0)
            plsc.addupdate_scatter(
                hist_v, [r >> 3, r & 7, cols], ones, mask=tok > 0
            )

    HALF = RT // 2
    top = pltpu.async_copy(
        hist_v.at[pl.ds(0, HALF)],
        out_hbm.at[pl.ds(0, HALF), ct, :, pl.ds(ci0, CPW)],
        out_sem,
    )
    pltpu.sync_copy(
        hist_v.at[pl.ds(HALF, RT - HALF)],
        out_hbm.at[pl.ds(HALF, RT - HALF), ct, :, pl.ds(ci0, CPW)],
    )
    top.wait()


def kernel(inputs):
    out4 = _bow_kernel(inputs.T)
    x = out4.transpose(0, 2, 1, 3).reshape(RT * 8, B)
    return x.T[:, :OUT_W]
